# baseline (speedup 1.0000x reference)
import functools

import jax
import jax.numpy as jnp
from jax import lax
from jax.experimental import pallas as pl
from jax.experimental.pallas import tpu as pltpu
from jax.experimental.pallas import tpu_sc as plsc

B, S = 4096, 50
SP = 56
D = 256
O = 64
NC, NS = 2, 16
NW = NC * NS
ROWS_PER_W = B // NW
NB_ROWS = 4
NBLK = ROWS_PER_W // NB_ROWS


@functools.partial(
    pl.kernel,
    out_type=jax.ShapeDtypeStruct((B, SP, D), jnp.float32),
    mesh=plsc.VectorSubcoreMesh(core_axis_name="c", subcore_axis_name="s"),
    scratch_types=[
        pltpu.VMEM((ROWS_PER_W, SP), jnp.int32),
        pltpu.VMEM((NB_ROWS, SP, D), jnp.float32),
        pltpu.VMEM((NB_ROWS, SP, D), jnp.float32),
        pltpu.SemaphoreType.DMA,
        pltpu.SemaphoreType.DMA,
        pltpu.SemaphoreType.DMA,
        pltpu.SemaphoreType.DMA,
    ],
)
def _sc_gather(table_hbm, x_hbm, h_hbm, idx_v, buf_a, buf_b,
               gsem_a, gsem_b, osem_a, osem_b):
    wid = lax.axis_index("s") * NC + lax.axis_index("c")
    r0 = wid * ROWS_PER_W
    pltpu.sync_copy(x_hbm.at[pl.ds(r0, ROWS_PER_W), :], idx_v)

    def start_gathers(blk, buf, gsem):
        for j in range(NB_ROWS):
            pltpu.async_copy(
                table_hbm.at[idx_v.at[blk * NB_ROWS + j, :]],
                buf.at[j], gsem)

    def wait_gathers(blk, buf, gsem):
        for j in range(NB_ROWS):
            pltpu.make_async_copy(
                table_hbm.at[idx_v.at[blk * NB_ROWS + j, :]],
                buf.at[j], gsem).wait()

    def out_slice(blk):
        return h_hbm.at[pl.ds(r0 + blk * NB_ROWS, NB_ROWS), :, :]

    def start_out(blk, buf, osem):
        pltpu.async_copy(buf, out_slice(blk), osem)

    def wait_out(blk, buf, osem):
        pltpu.make_async_copy(buf, out_slice(blk), osem).wait()

    start_gathers(0, buf_a, gsem_a)

    def body(k, carry):
        blk_a = 2 * k
        blk_b = 2 * k + 1

        @pl.when(k > 0)
        def _():
            wait_out(blk_b - 2, buf_b, osem_b)
        start_gathers(blk_b, buf_b, gsem_b)
        wait_gathers(blk_a, buf_a, gsem_a)
        start_out(blk_a, buf_a, osem_a)

        @pl.when(k < NBLK // 2 - 1)
        def _():
            wait_out(blk_a, buf_a, osem_a)
            start_gathers(blk_a + 2, buf_a, gsem_a)
        wait_gathers(blk_b, buf_b, gsem_b)
        start_out(blk_b, buf_b, osem_b)
        return carry

    lax.fori_loop(0, NBLK // 2, body, 0)
    wait_out(NBLK - 2, buf_a, osem_a)
    wait_out(NBLK - 1, buf_b, osem_b)


BLK_B = 64


def _tc_matmul_body(h_ref, w_ref, b_ref, o_ref):
    h2 = h_ref[...].reshape(BLK_B * SP, D)
    acc = lax.dot_general(
        h2, w_ref[...], (((1,), (1,)), ((), ())),
        preferred_element_type=jnp.float32,
    ) + b_ref[...]
    o_ref[...] = acc.reshape(BLK_B, SP, O)


def _tc_matmul(h, W, b):
    return pl.pallas_call(
        _tc_matmul_body,
        grid=(B // BLK_B,),
        in_specs=[
            pl.BlockSpec((BLK_B, SP, D), lambda i: (i, 0, 0)),
            pl.BlockSpec((O, D), lambda i: (0, 0)),
            pl.BlockSpec((1, O), lambda i: (0, 0)),
        ],
        out_specs=pl.BlockSpec((BLK_B, SP, O), lambda i: (i, 0, 0)),
        out_shape=jax.ShapeDtypeStruct((B, S, O), jnp.float32),
    )(h, W, b.reshape(1, O))


def kernel(x, table, W, b):
    xp = jnp.pad(x.astype(jnp.int32), ((0, 0), (0, SP - S)))
    h = _sc_gather(table, xp)
    return _tc_matmul(h, W, b)
